# baseline (speedup 1.0000x reference)
import functools

import numpy as np
import jax
import jax.numpy as jnp
from jax import lax
from jax.experimental import pallas as pl
from jax.experimental.pallas import tpu as pltpu
from jax.experimental.pallas import tpu_sc as plsc

_N = 10000
_E = 160000
_F_IN = 32
_F_EDGE = 16
_C_OUT = 16
_EDGE_H = 64
_D1 = 64
_D2 = 8
_G = 64

_NW = 32
_CH = 128
_CPW = 40
_EP = _NW * _CPW * _CH
_EB = 2048
_GRID = _EP // _EB

_DP = lax.Precision.DEFAULT

def _sc_gather(node_attr, batching, src2d, dst2d):
    mesh = plsc.VectorSubcoreMesh(core_axis_name="c", subcore_axis_name="s")

    @functools.partial(
        pl.kernel,
        mesh=mesh,
        compiler_params=pltpu.CompilerParams(use_tc_tiling_on_sc=False),
        out_type=[
            jax.ShapeDtypeStruct((_EP, _F_IN), jnp.float32),
            jax.ShapeDtypeStruct((_EP, 16), jnp.int32),
        ],
        scratch_types=[
            pltpu.VMEM((_CH,), jnp.int32),
            pltpu.VMEM((_CH, _F_IN), jnp.float32),
            pltpu.VMEM((_CH, 16), jnp.int32),
            pltpu.SemaphoreType.DMA,
        ],
    )
    def k(na_hbm, bat_hbm, src_hbm, dst_hbm, xj_hbm, gb_hbm,
          idx_v, rows_v, gbrows_v, sem):
        wid = lax.axis_index("s") * 2 + lax.axis_index("c")

        def body(c, carry):
            row = wid * _CPW + c
            pltpu.sync_copy(src_hbm.at[row], idx_v)
            pltpu.async_copy(na_hbm.at[idx_v], rows_v, sem).wait()
            pltpu.sync_copy(rows_v, xj_hbm.at[pl.ds(row * _CH, _CH)])
            pltpu.sync_copy(dst_hbm.at[row], idx_v)
            pltpu.async_copy(bat_hbm.at[idx_v], gbrows_v, sem).wait()
            pltpu.sync_copy(gbrows_v, gb_hbm.at[pl.ds(row * _CH, _CH)])
            return carry

        lax.fori_loop(0, _CPW, body, 0)

    return k(node_attr, batching, src2d, dst2d)


def _tc_body(ea_ref, xj_ref, gb_ref, na_ref, bat_ref,
             W1_ref, b1_ref, W2_ref, b2_ref, Wr_ref, bc_ref,
             Wd1_ref, bd1_ref, Wd2_ref, bd2_ref,
             out_ref, acc_ref):
    i = pl.program_id(0)

    @pl.when(i == 0)
    def _init():
        ohB = (bat_ref[...] == lax.broadcasted_iota(
            jnp.int32, (1, _G), 1)).astype(jnp.float32)
        t = lax.dot_general(ohB, na_ref[...],
                            (((0,), (0,)), ((), ())), precision=_DP)
        gnode = jnp.dot(t, Wr_ref[...], precision=_DP)
        cnt = lax.dot_general(ohB, jnp.ones((_N, 1), jnp.float32),
                              (((0,), (0,)), ((), ())), precision=_DP)
        acc_ref[...] = gnode + cnt * bc_ref[...]

    h = jnp.maximum(
        jnp.dot(ea_ref[...], W1_ref[...], precision=_DP) + b1_ref[...], 0.0)
    We = jnp.dot(h, W2_ref[...], precision=_DP) + b2_ref[...]
    xj = xj_ref[...]
    msg = xj[:, 0:1] * We[:, 0:_C_OUT]
    for k in range(1, _F_IN):
        msg = msg + xj[:, k:k + 1] * We[:, k * _C_OUT:(k + 1) * _C_OUT]

    eid = i * _EB + lax.broadcasted_iota(jnp.int32, (_EB, 1), 0)
    valid = eid < _E
    oh = ((gb_ref[:, 0:1] == lax.broadcasted_iota(jnp.int32, (1, _G), 1))
          & valid).astype(jnp.float32)
    acc_ref[...] += lax.dot_general(oh, msg,
                                    (((0,), (0,)), ((), ())), precision=_DP)

    @pl.when(i == _GRID - 1)
    def _fin():
        g = acc_ref[...]
        gr = jnp.maximum(
            jnp.dot(g, Wd1_ref[...], precision=_DP) + bd1_ref[...], 0.0)
        out_ref[...] = jnp.dot(gr, Wd2_ref[...], precision=_DP) + bd2_ref[...]


def _tc_main(ea_p, xj, gb2, node_attr, bat2,
             W1, b1, W2, b2, W_root, b_conv, Wd1, bd1, Wd2, bd2):
    full = lambda shape: pl.BlockSpec(shape, lambda i: (0,) * len(shape))
    return pl.pallas_call(
        _tc_body,
        grid=(_GRID,),
        in_specs=[
            pl.BlockSpec((_EB, _F_EDGE), lambda i: (i, 0)),
            pl.BlockSpec((_EB, _F_IN), lambda i: (i, 0)),
            pl.BlockSpec((_EB, 16), lambda i: (i, 0)),
            full((_N, _F_IN)),
            full((_N, 1)),
            full((_F_EDGE, _EDGE_H)),
            full((1, _EDGE_H)),
            full((_EDGE_H, _F_IN * _C_OUT)),
            full((1, _F_IN * _C_OUT)),
            full((_F_IN, _C_OUT)),
            full((1, _C_OUT)),
            full((_C_OUT, _D1)),
            full((1, _D1)),
            full((_D1, _D2)),
            full((1, _D2)),
        ],
        out_specs=full((_G, _D2)),
        out_shape=jax.ShapeDtypeStruct((_G, _D2), jnp.float32),
        scratch_shapes=[pltpu.VMEM((_G, _C_OUT), jnp.float32)],
    )(ea_p, xj, gb2, node_attr, bat2,
      W1, b1, W2, b2, W_root, b_conv, Wd1, bd1, Wd2, bd2)


def kernel(node_attr, edge_index, edge_attr, batching, W1, b1, W2, b2,
           W_root, b_conv, Wd1, bd1, Wd2, bd2):
    pad = _EP - _E
    src2d = jnp.concatenate(
        [edge_index[0], jnp.zeros((pad,), jnp.int32)]).reshape(_NW * _CPW, _CH)
    dst2d = jnp.concatenate(
        [edge_index[1], jnp.zeros((pad,), jnp.int32)]).reshape(_NW * _CPW, _CH)
    ea_p = jnp.concatenate(
        [edge_attr, jnp.zeros((pad, _F_EDGE), jnp.float32)])

    bat16 = jnp.broadcast_to(batching[:, None], (_N, 16))
    xj, gb = _sc_gather(node_attr, bat16, src2d, dst2d)

    return _tc_main(
        ea_p, xj, gb, node_attr, batching.reshape(_N, 1),
        W1, b1.reshape(1, -1), W2, b2.reshape(1, -1),
        W_root, b_conv.reshape(1, -1), Wd1, bd1.reshape(1, -1),
        Wd2, bd2.reshape(1, -1))


# speedup vs baseline: 3.6492x; 3.6492x over previous
import functools

import numpy as np
import jax
import jax.numpy as jnp
from jax import lax
from jax.experimental import pallas as pl
from jax.experimental.pallas import tpu as pltpu
from jax.experimental.pallas import tpu_sc as plsc

_N = 10000
_E = 160000
_F_IN = 32
_F_EDGE = 16
_C_OUT = 16
_EDGE_H = 64
_D1 = 64
_D2 = 8
_G = 64

_NW = 32
_CH = 128
_CPW = 40
_EP = _NW * _CPW * _CH
_EB = 2048
_GRID = _EP // _EB

_DP = lax.Precision.DEFAULT

def _sc_gather(node_attr, batching, sd2d):
    mesh = plsc.VectorSubcoreMesh(core_axis_name="c", subcore_axis_name="s")

    @functools.partial(
        pl.kernel,
        mesh=mesh,
        compiler_params=pltpu.CompilerParams(use_tc_tiling_on_sc=False),
        out_type=[
            jax.ShapeDtypeStruct((_EP, _F_IN), jnp.float32),
            jax.ShapeDtypeStruct((_EP, 16), jnp.int32),
        ],
        scratch_types=[
            pltpu.VMEM((2, 2 * _CH), jnp.int32),
            pltpu.VMEM((2, _CH, _F_IN), jnp.float32),
            pltpu.VMEM((2, _CH, 16), jnp.int32),
            pltpu.SemaphoreType.DMA,
            pltpu.SemaphoreType.DMA,
        ],
    )
    def k(na_hbm, bat_hbm, sd_hbm, xj_hbm, gb_hbm,
          idx_v, rows_v, gbrows_v, semx, semg):
        wid = lax.axis_index("s") * 2 + lax.axis_index("c")

        def body(c, carry):
            cx = []
            cg = []
            for b in range(2):
                row = wid * _CPW + 2 * c + b
                pltpu.sync_copy(sd_hbm.at[row], idx_v.at[b])
                cx.append(pltpu.async_copy(
                    na_hbm.at[idx_v.at[b, pl.ds(0, _CH)]], rows_v.at[b], semx))
                cg.append(pltpu.async_copy(
                    bat_hbm.at[idx_v.at[b, pl.ds(_CH, _CH)]], gbrows_v.at[b],
                    semg))
            for b in range(2):
                row = wid * _CPW + 2 * c + b
                cx[b].wait()
                pltpu.sync_copy(rows_v.at[b], xj_hbm.at[pl.ds(row * _CH, _CH)])
                cg[b].wait()
                pltpu.sync_copy(gbrows_v.at[b], gb_hbm.at[pl.ds(row * _CH, _CH)])
            return carry

        lax.fori_loop(0, _CPW // 2, body, 0)

    return k(node_attr, batching, sd2d)


def _tc_body(ea_ref, xj_ref, gb_ref, na_ref, bat_ref,
             W1_ref, b1_ref, W2_ref, b2_ref, Wr_ref, bc_ref,
             Wd1_ref, bd1_ref, Wd2_ref, bd2_ref, R_ref, S_ref,
             out_ref, acc_ref):
    i = pl.program_id(0)

    @pl.when(i == 0)
    def _init():
        ohB = (bat_ref[...] == lax.broadcasted_iota(
            jnp.int32, (1, _G), 1)).astype(jnp.float32)
        t = lax.dot_general(ohB, na_ref[...],
                            (((0,), (0,)), ((), ())), precision=_DP)
        gnode = jnp.dot(t, Wr_ref[...], precision=_DP)
        cnt = lax.dot_general(ohB, jnp.ones((_N, 1), jnp.float32),
                              (((0,), (0,)), ((), ())), precision=_DP)
        acc_ref[...] = gnode + cnt * bc_ref[...]

    h = jnp.maximum(
        jnp.dot(ea_ref[...], W1_ref[...], precision=_DP) + b1_ref[...], 0.0)
    We = jnp.dot(h, W2_ref[...], precision=_DP) + b2_ref[...]
    xr = jnp.dot(xj_ref[...], R_ref[...], precision=_DP)
    msg = jnp.dot(xr * We, S_ref[...], precision=_DP)

    eid = i * _EB + lax.broadcasted_iota(jnp.int32, (_EB, 1), 0)
    valid = eid < _E
    oh = ((gb_ref[:, 0:1] == lax.broadcasted_iota(jnp.int32, (1, _G), 1))
          & valid).astype(jnp.float32)
    acc_ref[...] += lax.dot_general(oh, msg,
                                    (((0,), (0,)), ((), ())), precision=_DP)

    @pl.when(i == _GRID - 1)
    def _fin():
        g = acc_ref[...]
        gr = jnp.maximum(
            jnp.dot(g, Wd1_ref[...], precision=_DP) + bd1_ref[...], 0.0)
        out_ref[...] = jnp.dot(gr, Wd2_ref[...], precision=_DP) + bd2_ref[...]


def _tc_main(ea_p, xj, gb2, node_attr, bat2,
             W1, b1, W2, b2, W_root, b_conv, Wd1, bd1, Wd2, bd2, R, S):
    full = lambda shape: pl.BlockSpec(shape, lambda i: (0,) * len(shape))
    return pl.pallas_call(
        _tc_body,
        grid=(_GRID,),
        in_specs=[
            pl.BlockSpec((_EB, _F_EDGE), lambda i: (i, 0)),
            pl.BlockSpec((_EB, _F_IN), lambda i: (i, 0)),
            pl.BlockSpec((_EB, 16), lambda i: (i, 0)),
            full((_N, _F_IN)),
            full((_N, 1)),
            full((_F_EDGE, _EDGE_H)),
            full((1, _EDGE_H)),
            full((_EDGE_H, _F_IN * _C_OUT)),
            full((1, _F_IN * _C_OUT)),
            full((_F_IN, _C_OUT)),
            full((1, _C_OUT)),
            full((_C_OUT, _D1)),
            full((1, _D1)),
            full((_D1, _D2)),
            full((1, _D2)),
            full((_F_IN, _F_IN * _C_OUT)),
            full((_F_IN * _C_OUT, _C_OUT)),
        ],
        out_specs=full((_G, _D2)),
        out_shape=jax.ShapeDtypeStruct((_G, _D2), jnp.float32),
        scratch_shapes=[pltpu.VMEM((_G, _C_OUT), jnp.float32)],
    )(ea_p, xj, gb2, node_attr, bat2,
      W1, b1, W2, b2, W_root, b_conv, Wd1, bd1, Wd2, bd2, R, S)


def kernel(node_attr, edge_index, edge_attr, batching, W1, b1, W2, b2,
           W_root, b_conv, Wd1, bd1, Wd2, bd2):
    pad = _EP - _E
    src2d = jnp.concatenate(
        [edge_index[0], jnp.zeros((pad,), jnp.int32)]).reshape(_NW * _CPW, _CH)
    dst2d = jnp.concatenate(
        [edge_index[1], jnp.zeros((pad,), jnp.int32)]).reshape(_NW * _CPW, _CH)
    ea_p = jnp.concatenate(
        [edge_attr, jnp.zeros((pad, _F_EDGE), jnp.float32)])

    bat16 = jnp.broadcast_to(batching[:, None], (_N, 16))
    sd2d = jnp.concatenate([src2d, dst2d], axis=1)
    xj, gb = _sc_gather(node_attr, bat16, sd2d)

    R_np = np.zeros((_F_IN, _F_IN * _C_OUT), np.float32)
    for i in range(_F_IN):
        R_np[i, i * _C_OUT:(i + 1) * _C_OUT] = 1.0
    S_np = np.kron(np.ones((_F_IN, 1), np.float32), np.eye(_C_OUT, dtype=np.float32))

    return _tc_main(
        ea_p, xj, gb, node_attr, batching.reshape(_N, 1),
        W1, b1.reshape(1, -1), W2, b2.reshape(1, -1),
        W_root, b_conv.reshape(1, -1), Wd1, bd1.reshape(1, -1),
        Wd2, bd2.reshape(1, -1), jnp.asarray(R_np), jnp.asarray(S_np))


# speedup vs baseline: 3.9463x; 1.0814x over previous
import functools

import numpy as np
import jax
import jax.numpy as jnp
from jax import lax
from jax.experimental import pallas as pl
from jax.experimental.pallas import tpu as pltpu
from jax.experimental.pallas import tpu_sc as plsc

_N = 10000
_E = 160000
_F_IN = 32
_F_EDGE = 16
_C_OUT = 16
_EDGE_H = 64
_D1 = 64
_D2 = 8
_G = 64

_NW = 32
_CH = 128
_CPW = 40
_EP = _NW * _CPW * _CH
_NH = 2
_CPWH = _CPW // _NH
_EPH = _EP // _NH
_EB = 2048
_GRIDH = _EPH // _EB

_DP = lax.Precision.DEFAULT


def _sc_gather(cpw, node_attr, batching, sd2d):
    mesh = plsc.VectorSubcoreMesh(core_axis_name="c", subcore_axis_name="s")
    ep = _NW * cpw * _CH

    @functools.partial(
        pl.kernel,
        mesh=mesh,
        compiler_params=pltpu.CompilerParams(use_tc_tiling_on_sc=False),
        out_type=[
            jax.ShapeDtypeStruct((ep, _F_IN), jnp.float32),
            jax.ShapeDtypeStruct((ep, 16), jnp.int32),
        ],
        scratch_types=[
            pltpu.VMEM((2, 2 * _CH), jnp.int32),
            pltpu.VMEM((2, _CH, _F_IN), jnp.float32),
            pltpu.VMEM((2, _CH, 16), jnp.int32),
            pltpu.SemaphoreType.DMA,
            pltpu.SemaphoreType.DMA,
        ],
    )
    def k(na_hbm, bat_hbm, sd_hbm, xj_hbm, gb_hbm,
          idx_v, rows_v, gbrows_v, semx, semg):
        wid = lax.axis_index("s") * 2 + lax.axis_index("c")

        def body(c, carry):
            cx = []
            cg = []
            for b in range(2):
                row = wid * cpw + 2 * c + b
                pltpu.sync_copy(sd_hbm.at[row], idx_v.at[b])
                cx.append(pltpu.async_copy(
                    na_hbm.at[idx_v.at[b, pl.ds(0, _CH)]], rows_v.at[b], semx))
                cg.append(pltpu.async_copy(
                    bat_hbm.at[idx_v.at[b, pl.ds(_CH, _CH)]], gbrows_v.at[b],
                    semg))
            for b in range(2):
                row = wid * cpw + 2 * c + b
                cx[b].wait()
                pltpu.sync_copy(rows_v.at[b], xj_hbm.at[pl.ds(row * _CH, _CH)])
                cg[b].wait()
                pltpu.sync_copy(gbrows_v.at[b], gb_hbm.at[pl.ds(row * _CH, _CH)])
            return carry

        lax.fori_loop(0, cpw // 2, body, 0)

    return k(node_attr, batching, sd2d)


def _make_tc_edge(valid_limit):
    def body(ea_ref, xj_ref, gb_ref, W1_ref, b1_ref, W2_ref, b2_ref,
             R_ref, S_ref, out_ref):
        i = pl.program_id(0)

        @pl.when(i == 0)
        def _init():
            out_ref[...] = jnp.zeros((_G, _C_OUT), jnp.float32)

        h = jnp.maximum(
            jnp.dot(ea_ref[...], W1_ref[...], precision=_DP) + b1_ref[...],
            0.0)
        We = jnp.dot(h, W2_ref[...], precision=_DP) + b2_ref[...]
        xr = jnp.dot(xj_ref[...], R_ref[...], precision=_DP)
        msg = jnp.dot(xr * We, S_ref[...], precision=_DP)

        ohb = gb_ref[:, 0:1] == lax.broadcasted_iota(jnp.int32, (1, _G), 1)
        if valid_limit < _EPH:
            eid = i * _EB + lax.broadcasted_iota(jnp.int32, (_EB, 1), 0)
            ohb = ohb & (eid < valid_limit)
        oh = ohb.astype(jnp.float32)
        out_ref[...] += lax.dot_general(oh, msg,
                                        (((0,), (0,)), ((), ())),
                                        precision=_DP)
    return body


def _tc_edge(valid_limit, ea_h, xj_h, gb_h, W1, b1, W2, b2, R, S):
    full = lambda shape: pl.BlockSpec(shape, lambda i: (0,) * len(shape))
    return pl.pallas_call(
        _make_tc_edge(valid_limit),
        grid=(_GRIDH,),
        in_specs=[
            pl.BlockSpec((_EB, _F_EDGE), lambda i: (i, 0)),
            pl.BlockSpec((_EB, _F_IN), lambda i: (i, 0)),
            pl.BlockSpec((_EB, 16), lambda i: (i, 0)),
            full((_F_EDGE, _EDGE_H)),
            full((1, _EDGE_H)),
            full((_EDGE_H, _F_IN * _C_OUT)),
            full((1, _F_IN * _C_OUT)),
            full((_F_IN, _F_IN * _C_OUT)),
            full((_F_IN * _C_OUT, _C_OUT)),
        ],
        out_specs=full((_G, _C_OUT)),
        out_shape=jax.ShapeDtypeStruct((_G, _C_OUT), jnp.float32),
    )(ea_h, xj_h, gb_h, W1, b1, W2, b2, R, S)


def _tc_final_body(p0_ref, p1_ref, na_ref, bat_ref, Wr_ref, bc_ref,
                   Wd1_ref, bd1_ref, Wd2_ref, bd2_ref, out_ref):
    ohB = (bat_ref[...] == lax.broadcasted_iota(
        jnp.int32, (1, _G), 1)).astype(jnp.float32)
    t = lax.dot_general(ohB, na_ref[...],
                        (((0,), (0,)), ((), ())), precision=_DP)
    gnode = jnp.dot(t, Wr_ref[...], precision=_DP)
    cnt = lax.dot_general(ohB, jnp.ones((_N, 1), jnp.float32),
                          (((0,), (0,)), ((), ())), precision=_DP)
    g = p0_ref[...] + p1_ref[...] + gnode + cnt * bc_ref[...]
    gr = jnp.maximum(
        jnp.dot(g, Wd1_ref[...], precision=_DP) + bd1_ref[...], 0.0)
    out_ref[...] = jnp.dot(gr, Wd2_ref[...], precision=_DP) + bd2_ref[...]


def _tc_final(p0, p1, node_attr, bat2, W_root, b_conv, Wd1, bd1, Wd2, bd2):
    return pl.pallas_call(
        _tc_final_body,
        out_shape=jax.ShapeDtypeStruct((_G, _D2), jnp.float32),
    )(p0, p1, node_attr, bat2, W_root, b_conv, Wd1, bd1, Wd2, bd2)


def kernel(node_attr, edge_index, edge_attr, batching, W1, b1, W2, b2,
           W_root, b_conv, Wd1, bd1, Wd2, bd2):
    pad = _EP - _E
    src2d = jnp.concatenate(
        [edge_index[0], jnp.zeros((pad,), jnp.int32)]).reshape(_NW * _CPW, _CH)
    dst2d = jnp.concatenate(
        [edge_index[1], jnp.zeros((pad,), jnp.int32)]).reshape(_NW * _CPW, _CH)
    sd2d = jnp.concatenate([src2d, dst2d], axis=1)
    ea_p = jnp.concatenate(
        [edge_attr, jnp.zeros((pad, _F_EDGE), jnp.float32)])
    bat16 = jnp.broadcast_to(batching[:, None], (_N, 16))

    R_np = np.zeros((_F_IN, _F_IN * _C_OUT), np.float32)
    for i in range(_F_IN):
        R_np[i, i * _C_OUT:(i + 1) * _C_OUT] = 1.0
    S_np = np.kron(np.ones((_F_IN, 1), np.float32),
                   np.eye(_C_OUT, dtype=np.float32))
    R = jnp.asarray(R_np)
    S = jnp.asarray(S_np)

    b1r = b1.reshape(1, -1)
    b2r = b2.reshape(1, -1)
    rows_h = _NW * _CPWH
    parts = []
    for hh in range(_NH):
        sd_h = lax.slice_in_dim(sd2d, hh * rows_h, (hh + 1) * rows_h, axis=0)
        xj_h, gb_h = _sc_gather(_CPWH, node_attr, bat16, sd_h)
        ea_h = lax.slice_in_dim(ea_p, hh * _EPH, (hh + 1) * _EPH, axis=0)
        limit = min(_EPH, max(0, _E - hh * _EPH))
        parts.append(_tc_edge(limit, ea_h, xj_h, gb_h, W1, b1r, W2, b2r,
                              R, S))

    return _tc_final(parts[0], parts[1], node_attr, batching.reshape(_N, 1),
                     W_root, b_conv.reshape(1, -1), Wd1, bd1.reshape(1, -1),
                     Wd2, bd2.reshape(1, -1))


# speedup vs baseline: 4.0529x; 1.0270x over previous
import functools

import numpy as np
import jax
import jax.numpy as jnp
from jax import lax
from jax.experimental import pallas as pl
from jax.experimental.pallas import tpu as pltpu
from jax.experimental.pallas import tpu_sc as plsc

_N = 10000
_E = 160000
_F_IN = 32
_F_EDGE = 16
_C_OUT = 16
_EDGE_H = 64
_D1 = 64
_D2 = 8
_G = 64

_NW = 32
_CH = 128
_CPW = 40
_EP = _NW * _CPW * _CH
_NH = 2
_CPWH = _CPW // _NH
_EPH = _EP // _NH
_EB = 4096
_GRIDH = _EPH // _EB

_DP = lax.Precision.DEFAULT


def _sc_gather(cpw, node_attr, batching, sd2d):
    mesh = plsc.VectorSubcoreMesh(core_axis_name="c", subcore_axis_name="s")
    ep = _NW * cpw * _CH

    @functools.partial(
        pl.kernel,
        mesh=mesh,
        compiler_params=pltpu.CompilerParams(use_tc_tiling_on_sc=False),
        out_type=[
            jax.ShapeDtypeStruct((ep, _F_IN), jnp.float32),
            jax.ShapeDtypeStruct((ep, 16), jnp.int32),
        ],
        scratch_types=[
            pltpu.VMEM((2, 2 * _CH), jnp.int32),
            pltpu.VMEM((2, _CH, _F_IN), jnp.float32),
            pltpu.VMEM((2, _CH, 16), jnp.int32),
            pltpu.SemaphoreType.DMA,
            pltpu.SemaphoreType.DMA,
        ],
    )
    def k(na_hbm, bat_hbm, sd_hbm, xj_hbm, gb_hbm,
          idx_v, rows_v, gbrows_v, semx, semg):
        wid = lax.axis_index("s") * 2 + lax.axis_index("c")

        def body(c, carry):
            cx = []
            cg = []
            for b in range(2):
                row = wid * cpw + 2 * c + b
                pltpu.sync_copy(sd_hbm.at[row], idx_v.at[b])
                cx.append(pltpu.async_copy(
                    na_hbm.at[idx_v.at[b, pl.ds(0, _CH)]], rows_v.at[b], semx))
                cg.append(pltpu.async_copy(
                    bat_hbm.at[idx_v.at[b, pl.ds(_CH, _CH)]], gbrows_v.at[b],
                    semg))
            for b in range(2):
                row = wid * cpw + 2 * c + b
                cx[b].wait()
                pltpu.sync_copy(rows_v.at[b], xj_hbm.at[pl.ds(row * _CH, _CH)])
                cg[b].wait()
                pltpu.sync_copy(gbrows_v.at[b], gb_hbm.at[pl.ds(row * _CH, _CH)])
            return carry

        lax.fori_loop(0, cpw // 2, body, 0)

    return k(node_attr, batching, sd2d)


def _make_tc_edge(valid_limit):
    def body(ea_ref, xj_ref, gb_ref, W1_ref, b1_ref, W2_ref, b2_ref,
             R_ref, S_ref, out_ref):
        i = pl.program_id(0)

        @pl.when(i == 0)
        def _init():
            out_ref[...] = jnp.zeros((_G, _C_OUT), jnp.float32)

        h = jnp.maximum(
            jnp.dot(ea_ref[...], W1_ref[...], precision=_DP) + b1_ref[...],
            0.0)
        We = jnp.dot(h, W2_ref[...], precision=_DP) + b2_ref[...]
        xr = jnp.dot(xj_ref[...], R_ref[...], precision=_DP)
        msg = jnp.dot(xr * We, S_ref[...], precision=_DP)

        ohb = gb_ref[:, 0:1] == lax.broadcasted_iota(jnp.int32, (1, _G), 1)
        if valid_limit < _EPH:
            eid = i * _EB + lax.broadcasted_iota(jnp.int32, (_EB, 1), 0)
            ohb = ohb & (eid < valid_limit)
        oh = ohb.astype(jnp.float32)
        out_ref[...] += lax.dot_general(oh, msg,
                                        (((0,), (0,)), ((), ())),
                                        precision=_DP)
    return body


def _tc_edge(valid_limit, ea_h, xj_h, gb_h, W1, b1, W2, b2, R, S):
    full = lambda shape: pl.BlockSpec(shape, lambda i: (0,) * len(shape))
    return pl.pallas_call(
        _make_tc_edge(valid_limit),
        grid=(_GRIDH,),
        in_specs=[
            pl.BlockSpec((_EB, _F_EDGE), lambda i: (i, 0)),
            pl.BlockSpec((_EB, _F_IN), lambda i: (i, 0)),
            pl.BlockSpec((_EB, 16), lambda i: (i, 0)),
            full((_F_EDGE, _EDGE_H)),
            full((1, _EDGE_H)),
            full((_EDGE_H, _F_IN * _C_OUT)),
            full((1, _F_IN * _C_OUT)),
            full((_F_IN, _F_IN * _C_OUT)),
            full((_F_IN * _C_OUT, _C_OUT)),
        ],
        out_specs=full((_G, _C_OUT)),
        out_shape=jax.ShapeDtypeStruct((_G, _C_OUT), jnp.float32),
    )(ea_h, xj_h, gb_h, W1, b1, W2, b2, R, S)


def _tc_final_body(p0_ref, p1_ref, na_ref, bat_ref, Wr_ref, bc_ref,
                   Wd1_ref, bd1_ref, Wd2_ref, bd2_ref, out_ref):
    ohB = (bat_ref[...] == lax.broadcasted_iota(
        jnp.int32, (1, _G), 1)).astype(jnp.float32)
    t = lax.dot_general(ohB, na_ref[...],
                        (((0,), (0,)), ((), ())), precision=_DP)
    gnode = jnp.dot(t, Wr_ref[...], precision=_DP)
    cnt = lax.dot_general(ohB, jnp.ones((_N, 1), jnp.float32),
                          (((0,), (0,)), ((), ())), precision=_DP)
    g = p0_ref[...] + p1_ref[...] + gnode + cnt * bc_ref[...]
    gr = jnp.maximum(
        jnp.dot(g, Wd1_ref[...], precision=_DP) + bd1_ref[...], 0.0)
    out_ref[...] = jnp.dot(gr, Wd2_ref[...], precision=_DP) + bd2_ref[...]


def _tc_final(p0, p1, node_attr, bat2, W_root, b_conv, Wd1, bd1, Wd2, bd2):
    return pl.pallas_call(
        _tc_final_body,
        out_shape=jax.ShapeDtypeStruct((_G, _D2), jnp.float32),
    )(p0, p1, node_attr, bat2, W_root, b_conv, Wd1, bd1, Wd2, bd2)


def kernel(node_attr, edge_index, edge_attr, batching, W1, b1, W2, b2,
           W_root, b_conv, Wd1, bd1, Wd2, bd2):
    pad = _EP - _E
    src2d = jnp.concatenate(
        [edge_index[0], jnp.zeros((pad,), jnp.int32)]).reshape(_NW * _CPW, _CH)
    dst2d = jnp.concatenate(
        [edge_index[1], jnp.zeros((pad,), jnp.int32)]).reshape(_NW * _CPW, _CH)
    sd2d = jnp.concatenate([src2d, dst2d], axis=1)
    ea_p = jnp.concatenate(
        [edge_attr, jnp.zeros((pad, _F_EDGE), jnp.float32)])
    bat16 = jnp.broadcast_to(batching[:, None], (_N, 16))

    R_np = np.zeros((_F_IN, _F_IN * _C_OUT), np.float32)
    for i in range(_F_IN):
        R_np[i, i * _C_OUT:(i + 1) * _C_OUT] = 1.0
    S_np = np.kron(np.ones((_F_IN, 1), np.float32),
                   np.eye(_C_OUT, dtype=np.float32))
    R = jnp.asarray(R_np)
    S = jnp.asarray(S_np)

    b1r = b1.reshape(1, -1)
    b2r = b2.reshape(1, -1)
    rows_h = _NW * _CPWH
    parts = []
    for hh in range(_NH):
        sd_h = lax.slice_in_dim(sd2d, hh * rows_h, (hh + 1) * rows_h, axis=0)
        xj_h, gb_h = _sc_gather(_CPWH, node_attr, bat16, sd_h)
        ea_h = lax.slice_in_dim(ea_p, hh * _EPH, (hh + 1) * _EPH, axis=0)
        limit = min(_EPH, max(0, _E - hh * _EPH))
        parts.append(_tc_edge(limit, ea_h, xj_h, gb_h, W1, b1r, W2, b2r,
                              R, S))

    return _tc_final(parts[0], parts[1], node_attr, batching.reshape(_N, 1),
                     W_root, b_conv.reshape(1, -1), Wd1, bd1.reshape(1, -1),
                     Wd2, bd2.reshape(1, -1))


# speedup vs baseline: 4.1375x; 1.0209x over previous
import functools

import numpy as np
import jax
import jax.numpy as jnp
from jax import lax
from jax.experimental import pallas as pl
from jax.experimental.pallas import tpu as pltpu
from jax.experimental.pallas import tpu_sc as plsc

_N = 10000
_E = 160000
_F_IN = 32
_F_EDGE = 16
_C_OUT = 16
_EDGE_H = 64
_D1 = 64
_D2 = 8
_G = 64

_NW = 32
_CH = 128
_CPW = 40
_EP = _NW * _CPW * _CH
_NH = 2
_CPWH = _CPW // _NH
_EPH = _EP // _NH
_EB = 4096
_GRIDH = _EPH // _EB

_DP = lax.Precision.DEFAULT


def _sc_gather(cpw, node_attr, batching, sd2d):
    mesh = plsc.VectorSubcoreMesh(core_axis_name="c", subcore_axis_name="s")
    ep = _NW * cpw * _CH

    @functools.partial(
        pl.kernel,
        mesh=mesh,
        compiler_params=pltpu.CompilerParams(use_tc_tiling_on_sc=False),
        out_type=[
            jax.ShapeDtypeStruct((ep, _F_IN), jnp.bfloat16),
            jax.ShapeDtypeStruct((ep, 8), jnp.int32),
        ],
        scratch_types=[
            pltpu.VMEM((2, 2 * _CH), jnp.int32),
            pltpu.VMEM((2, _CH, _F_IN), jnp.bfloat16),
            pltpu.VMEM((2, _CH, 8), jnp.int32),
            pltpu.SemaphoreType.DMA,
            pltpu.SemaphoreType.DMA,
        ],
    )
    def k(na_hbm, bat_hbm, sd_hbm, xj_hbm, gb_hbm,
          idx_v, rows_v, gbrows_v, semx, semg):
        wid = lax.axis_index("s") * 2 + lax.axis_index("c")

        def body(c, carry):
            cx = []
            cg = []
            for b in range(2):
                row = wid * cpw + 2 * c + b
                pltpu.sync_copy(sd_hbm.at[row], idx_v.at[b])
                cx.append(pltpu.async_copy(
                    na_hbm.at[idx_v.at[b, pl.ds(0, _CH)]], rows_v.at[b], semx))
                cg.append(pltpu.async_copy(
                    bat_hbm.at[idx_v.at[b, pl.ds(_CH, _CH)]], gbrows_v.at[b],
                    semg))
            for b in range(2):
                row = wid * cpw + 2 * c + b
                cx[b].wait()
                pltpu.sync_copy(rows_v.at[b], xj_hbm.at[pl.ds(row * _CH, _CH)])
                cg[b].wait()
                pltpu.sync_copy(gbrows_v.at[b], gb_hbm.at[pl.ds(row * _CH, _CH)])
            return carry

        lax.fori_loop(0, cpw // 2, body, 0)

    return k(node_attr, batching, sd2d)


def _make_tc_edge(valid_limit):
    def body(ea_ref, xj_ref, gb_ref, W1_ref, b1_ref, W2_ref, b2_ref,
             R_ref, S_ref, out_ref):
        i = pl.program_id(0)

        @pl.when(i == 0)
        def _init():
            out_ref[...] = jnp.zeros((_G, _C_OUT), jnp.float32)

        h = jnp.maximum(
            jnp.dot(ea_ref[...], W1_ref[...], precision=_DP) + b1_ref[...],
            0.0)
        We = jnp.dot(h, W2_ref[...], precision=_DP) + b2_ref[...]
        xr = jnp.dot(xj_ref[...].astype(jnp.float32), R_ref[...],
                     precision=_DP)
        msg = jnp.dot(xr * We, S_ref[...], precision=_DP)

        ohb = gb_ref[:, 0:1] == lax.broadcasted_iota(jnp.int32, (1, _G), 1)
        if valid_limit < _EPH:
            eid = i * _EB + lax.broadcasted_iota(jnp.int32, (_EB, 1), 0)
            ohb = ohb & (eid < valid_limit)
        oh = ohb.astype(jnp.float32)
        out_ref[...] += lax.dot_general(oh, msg,
                                        (((0,), (0,)), ((), ())),
                                        precision=_DP)
    return body


def _tc_edge(valid_limit, ea_h, xj_h, gb_h, W1, b1, W2, b2, R, S):
    full = lambda shape: pl.BlockSpec(shape, lambda i: (0,) * len(shape))
    return pl.pallas_call(
        _make_tc_edge(valid_limit),
        grid=(_GRIDH,),
        in_specs=[
            pl.BlockSpec((_EB, _F_EDGE), lambda i: (i, 0)),
            pl.BlockSpec((_EB, _F_IN), lambda i: (i, 0)),
            pl.BlockSpec((_EB, 8), lambda i: (i, 0)),
            full((_F_EDGE, _EDGE_H)),
            full((1, _EDGE_H)),
            full((_EDGE_H, _F_IN * _C_OUT)),
            full((1, _F_IN * _C_OUT)),
            full((_F_IN, _F_IN * _C_OUT)),
            full((_F_IN * _C_OUT, _C_OUT)),
        ],
        out_specs=full((_G, _C_OUT)),
        out_shape=jax.ShapeDtypeStruct((_G, _C_OUT), jnp.float32),
    )(ea_h, xj_h, gb_h, W1, b1, W2, b2, R, S)


def _tc_final_body(p0_ref, p1_ref, na_ref, bat_ref, Wr_ref, bc_ref,
                   Wd1_ref, bd1_ref, Wd2_ref, bd2_ref, out_ref):
    ohB = (bat_ref[...] == lax.broadcasted_iota(
        jnp.int32, (1, _G), 1)).astype(jnp.float32)
    t = lax.dot_general(ohB, na_ref[...],
                        (((0,), (0,)), ((), ())), precision=_DP)
    gnode = jnp.dot(t, Wr_ref[...], precision=_DP)
    cnt = lax.dot_general(ohB, jnp.ones((_N, 1), jnp.float32),
                          (((0,), (0,)), ((), ())), precision=_DP)
    g = p0_ref[...] + p1_ref[...] + gnode + cnt * bc_ref[...]
    gr = jnp.maximum(
        jnp.dot(g, Wd1_ref[...], precision=_DP) + bd1_ref[...], 0.0)
    out_ref[...] = jnp.dot(gr, Wd2_ref[...], precision=_DP) + bd2_ref[...]


def _tc_final(p0, p1, node_attr, bat2, W_root, b_conv, Wd1, bd1, Wd2, bd2):
    return pl.pallas_call(
        _tc_final_body,
        out_shape=jax.ShapeDtypeStruct((_G, _D2), jnp.float32),
    )(p0, p1, node_attr, bat2, W_root, b_conv, Wd1, bd1, Wd2, bd2)


def kernel(node_attr, edge_index, edge_attr, batching, W1, b1, W2, b2,
           W_root, b_conv, Wd1, bd1, Wd2, bd2):
    pad = _EP - _E
    src2d = jnp.concatenate(
        [edge_index[0], jnp.zeros((pad,), jnp.int32)]).reshape(_NW * _CPW, _CH)
    dst2d = jnp.concatenate(
        [edge_index[1], jnp.zeros((pad,), jnp.int32)]).reshape(_NW * _CPW, _CH)
    sd2d = jnp.concatenate([src2d, dst2d], axis=1)
    ea_p = jnp.concatenate(
        [edge_attr, jnp.zeros((pad, _F_EDGE), jnp.float32)])
    bat8 = jnp.broadcast_to(batching[:, None], (_N, 8))
    nab = node_attr.astype(jnp.bfloat16)

    R_np = np.zeros((_F_IN, _F_IN * _C_OUT), np.float32)
    for i in range(_F_IN):
        R_np[i, i * _C_OUT:(i + 1) * _C_OUT] = 1.0
    S_np = np.kron(np.ones((_F_IN, 1), np.float32),
                   np.eye(_C_OUT, dtype=np.float32))
    R = jnp.asarray(R_np)
    S = jnp.asarray(S_np)

    b1r = b1.reshape(1, -1)
    b2r = b2.reshape(1, -1)
    rows_h = _NW * _CPWH
    parts = []
    for hh in range(_NH):
        sd_h = lax.slice_in_dim(sd2d, hh * rows_h, (hh + 1) * rows_h, axis=0)
        xj_h, gb_h = _sc_gather(_CPWH, nab, bat8, sd_h)
        ea_h = lax.slice_in_dim(ea_p, hh * _EPH, (hh + 1) * _EPH, axis=0)
        limit = min(_EPH, max(0, _E - hh * _EPH))
        parts.append(_tc_edge(limit, ea_h, xj_h, gb_h, W1, b1r, W2, b2r,
                              R, S))

    return _tc_final(parts[0], parts[1], node_attr, batching.reshape(_N, 1),
                     W_root, b_conv.reshape(1, -1), Wd1, bd1.reshape(1, -1),
                     Wd2, bd2.reshape(1, -1))


# speedup vs baseline: 4.1856x; 1.0116x over previous
import functools

import numpy as np
import jax
import jax.numpy as jnp
from jax import lax
from jax.experimental import pallas as pl
from jax.experimental.pallas import tpu as pltpu
from jax.experimental.pallas import tpu_sc as plsc

_N = 10000
_E = 160000
_F_IN = 32
_F_EDGE = 16
_C_OUT = 16
_EDGE_H = 64
_D1 = 64
_D2 = 8
_G = 64

_NW = 32
_CH = 128
_CPW = 40
_EP = _NW * _CPW * _CH
_NH = 4
_CPWH = _CPW // _NH
_EPH = _EP // _NH
_EB = 4096
_GRIDH = _EPH // _EB

_DP = lax.Precision.DEFAULT


def _sc_gather(cpw, node_attr, batching, sd2d):
    mesh = plsc.VectorSubcoreMesh(core_axis_name="c", subcore_axis_name="s")
    ep = _NW * cpw * _CH

    @functools.partial(
        pl.kernel,
        mesh=mesh,
        compiler_params=pltpu.CompilerParams(use_tc_tiling_on_sc=False),
        out_type=[
            jax.ShapeDtypeStruct((ep, _F_IN), jnp.bfloat16),
            jax.ShapeDtypeStruct((ep, 8), jnp.int32),
        ],
        scratch_types=[
            pltpu.VMEM((2, 2 * _CH), jnp.int32),
            pltpu.VMEM((2, _CH, _F_IN), jnp.bfloat16),
            pltpu.VMEM((2, _CH, 8), jnp.int32),
            pltpu.SemaphoreType.DMA,
            pltpu.SemaphoreType.DMA,
        ],
    )
    def k(na_hbm, bat_hbm, sd_hbm, xj_hbm, gb_hbm,
          idx_v, rows_v, gbrows_v, semx, semg):
        wid = lax.axis_index("s") * 2 + lax.axis_index("c")

        def body(c, carry):
            cx = []
            cg = []
            for b in range(2):
                row = wid * cpw + 2 * c + b
                pltpu.sync_copy(sd_hbm.at[row], idx_v.at[b])
                cx.append(pltpu.async_copy(
                    na_hbm.at[idx_v.at[b, pl.ds(0, _CH)]], rows_v.at[b], semx))
                cg.append(pltpu.async_copy(
                    bat_hbm.at[idx_v.at[b, pl.ds(_CH, _CH)]], gbrows_v.at[b],
                    semg))
            for b in range(2):
                row = wid * cpw + 2 * c + b
                cx[b].wait()
                pltpu.sync_copy(rows_v.at[b], xj_hbm.at[pl.ds(row * _CH, _CH)])
                cg[b].wait()
                pltpu.sync_copy(gbrows_v.at[b], gb_hbm.at[pl.ds(row * _CH, _CH)])
            return carry

        lax.fori_loop(0, cpw // 2, body, 0)

    return k(node_attr, batching, sd2d)


def _make_tc_edge(valid_limit):
    def body(ea_ref, xj_ref, gb_ref, W1_ref, b1_ref, W2_ref, b2_ref,
             R_ref, S_ref, out_ref):
        i = pl.program_id(0)

        @pl.when(i == 0)
        def _init():
            out_ref[...] = jnp.zeros((_G, _C_OUT), jnp.float32)

        h = jnp.maximum(
            jnp.dot(ea_ref[...], W1_ref[...], precision=_DP) + b1_ref[...],
            0.0)
        We = jnp.dot(h, W2_ref[...], precision=_DP) + b2_ref[...]
        xr = jnp.dot(xj_ref[...].astype(jnp.float32), R_ref[...],
                     precision=_DP)
        msg = jnp.dot(xr * We, S_ref[...], precision=_DP)

        ohb = gb_ref[:, 0:1] == lax.broadcasted_iota(jnp.int32, (1, _G), 1)
        if valid_limit < _EPH:
            eid = i * _EB + lax.broadcasted_iota(jnp.int32, (_EB, 1), 0)
            ohb = ohb & (eid < valid_limit)
        oh = ohb.astype(jnp.float32)
        out_ref[...] += lax.dot_general(oh, msg,
                                        (((0,), (0,)), ((), ())),
                                        precision=_DP)
    return body


def _tc_edge(valid_limit, ea_h, xj_h, gb_h, W1, b1, W2, b2, R, S):
    full = lambda shape: pl.BlockSpec(shape, lambda i: (0,) * len(shape))
    return pl.pallas_call(
        _make_tc_edge(valid_limit),
        grid=(_GRIDH,),
        in_specs=[
            pl.BlockSpec((_EB, _F_EDGE), lambda i: (i, 0)),
            pl.BlockSpec((_EB, _F_IN), lambda i: (i, 0)),
            pl.BlockSpec((_EB, 8), lambda i: (i, 0)),
            full((_F_EDGE, _EDGE_H)),
            full((1, _EDGE_H)),
            full((_EDGE_H, _F_IN * _C_OUT)),
            full((1, _F_IN * _C_OUT)),
            full((_F_IN, _F_IN * _C_OUT)),
            full((_F_IN * _C_OUT, _C_OUT)),
        ],
        out_specs=full((_G, _C_OUT)),
        out_shape=jax.ShapeDtypeStruct((_G, _C_OUT), jnp.float32),
    )(ea_h, xj_h, gb_h, W1, b1, W2, b2, R, S)


def _tc_final_body(*refs):
    p_refs = refs[:_NH]
    (na_ref, bat_ref, Wr_ref, bc_ref,
     Wd1_ref, bd1_ref, Wd2_ref, bd2_ref, out_ref) = refs[_NH:]
    ohB = (bat_ref[...] == lax.broadcasted_iota(
        jnp.int32, (1, _G), 1)).astype(jnp.float32)
    t = lax.dot_general(ohB, na_ref[...],
                        (((0,), (0,)), ((), ())), precision=_DP)
    gnode = jnp.dot(t, Wr_ref[...], precision=_DP)
    cnt = lax.dot_general(ohB, jnp.ones((_N, 1), jnp.float32),
                          (((0,), (0,)), ((), ())), precision=_DP)
    g = gnode + cnt * bc_ref[...]
    for p_ref in p_refs:
        g = g + p_ref[...]
    gr = jnp.maximum(
        jnp.dot(g, Wd1_ref[...], precision=_DP) + bd1_ref[...], 0.0)
    out_ref[...] = jnp.dot(gr, Wd2_ref[...], precision=_DP) + bd2_ref[...]


def _tc_final(parts, node_attr, bat2, W_root, b_conv, Wd1, bd1, Wd2, bd2):
    return pl.pallas_call(
        _tc_final_body,
        out_shape=jax.ShapeDtypeStruct((_G, _D2), jnp.float32),
    )(*parts, node_attr, bat2, W_root, b_conv, Wd1, bd1, Wd2, bd2)


def kernel(node_attr, edge_index, edge_attr, batching, W1, b1, W2, b2,
           W_root, b_conv, Wd1, bd1, Wd2, bd2):
    pad = _EP - _E
    src2d = jnp.concatenate(
        [edge_index[0], jnp.zeros((pad,), jnp.int32)]).reshape(_NW * _CPW, _CH)
    dst2d = jnp.concatenate(
        [edge_index[1], jnp.zeros((pad,), jnp.int32)]).reshape(_NW * _CPW, _CH)
    sd2d = jnp.concatenate([src2d, dst2d], axis=1)
    ea_p = jnp.concatenate(
        [edge_attr, jnp.zeros((pad, _F_EDGE), jnp.float32)])
    bat8 = jnp.broadcast_to(batching[:, None], (_N, 8))
    nab = node_attr.astype(jnp.bfloat16)

    R_np = np.zeros((_F_IN, _F_IN * _C_OUT), np.float32)
    for i in range(_F_IN):
        R_np[i, i * _C_OUT:(i + 1) * _C_OUT] = 1.0
    S_np = np.kron(np.ones((_F_IN, 1), np.float32),
                   np.eye(_C_OUT, dtype=np.float32))
    R = jnp.asarray(R_np)
    S = jnp.asarray(S_np)

    b1r = b1.reshape(1, -1)
    b2r = b2.reshape(1, -1)
    rows_h = _NW * _CPWH
    parts = []
    for hh in range(_NH):
        sd_h = lax.slice_in_dim(sd2d, hh * rows_h, (hh + 1) * rows_h, axis=0)
        xj_h, gb_h = _sc_gather(_CPWH, nab, bat8, sd_h)
        ea_h = lax.slice_in_dim(ea_p, hh * _EPH, (hh + 1) * _EPH, axis=0)
        limit = min(_EPH, max(0, _E - hh * _EPH))
        parts.append(_tc_edge(limit, ea_h, xj_h, gb_h, W1, b1r, W2, b2r,
                              R, S))

    return _tc_final(parts, node_attr, batching.reshape(_N, 1),
                     W_root, b_conv.reshape(1, -1), Wd1, bd1.reshape(1, -1),
                     Wd2, bd2.reshape(1, -1))


# speedup vs baseline: 4.2361x; 1.0121x over previous
import functools

import numpy as np
import jax
import jax.numpy as jnp
from jax import lax
from jax.experimental import pallas as pl
from jax.experimental.pallas import tpu as pltpu
from jax.experimental.pallas import tpu_sc as plsc

_N = 10000
_E = 160000
_F_IN = 32
_F_EDGE = 16
_C_OUT = 16
_EDGE_H = 64
_D1 = 64
_D2 = 8
_G = 64

_NW = 32
_CH = 128
_CPW = 40
_EP = _NW * _CPW * _CH
_NH = 4
_CPWH = _CPW // _NH
_EPH = _EP // _NH
_EB = 4096
_GRIDH = _EPH // _EB

_DP = lax.Precision.DEFAULT


def _sc_gather(cpw, node_attr, batching, sd2d):
    mesh = plsc.VectorSubcoreMesh(core_axis_name="c", subcore_axis_name="s")
    ep = _NW * cpw * _CH

    @functools.partial(
        pl.kernel,
        mesh=mesh,
        compiler_params=pltpu.CompilerParams(use_tc_tiling_on_sc=False),
        out_type=[
            jax.ShapeDtypeStruct((ep, _F_IN), jnp.bfloat16),
            jax.ShapeDtypeStruct((ep, 8), jnp.int32),
        ],
        scratch_types=[
            pltpu.VMEM((2, 2 * _CH), jnp.int32),
            pltpu.VMEM((2, _CH, _F_IN), jnp.bfloat16),
            pltpu.VMEM((2, _CH, 8), jnp.int32),
            pltpu.SemaphoreType.DMA,
            pltpu.SemaphoreType.DMA,
        ],
    )
    def k(na_hbm, bat_hbm, sd_hbm, xj_hbm, gb_hbm,
          idx_v, rows_v, gbrows_v, semx, semg):
        wid = lax.axis_index("s") * 2 + lax.axis_index("c")

        def body(c, carry):
            cx = []
            cg = []
            for b in range(2):
                row = wid * cpw + 2 * c + b
                pltpu.sync_copy(sd_hbm.at[row], idx_v.at[b])
                cx.append(pltpu.async_copy(
                    na_hbm.at[idx_v.at[b, pl.ds(0, _CH)]], rows_v.at[b], semx))
                cg.append(pltpu.async_copy(
                    bat_hbm.at[idx_v.at[b, pl.ds(_CH, _CH)]], gbrows_v.at[b],
                    semg))
            for b in range(2):
                row = wid * cpw + 2 * c + b
                cx[b].wait()
                pltpu.sync_copy(rows_v.at[b], xj_hbm.at[pl.ds(row * _CH, _CH)])
                cg[b].wait()
                pltpu.sync_copy(gbrows_v.at[b], gb_hbm.at[pl.ds(row * _CH, _CH)])
            return carry

        lax.fori_loop(0, cpw // 2, body, 0)

    return k(node_attr, batching, sd2d)


def _make_tc_edge(valid_limit):
    def body(ea_ref, xj_ref, gb_ref, W1_ref, b1_ref, W2_ref, b2_ref,
             R_ref, S_ref, out_ref):
        i = pl.program_id(0)

        @pl.when(i == 0)
        def _init():
            out_ref[...] = jnp.zeros((_G, _C_OUT), jnp.float32)

        mm = lambda a, b: lax.dot_general(
            a, b, (((1,), (0,)), ((), ())),
            preferred_element_type=jnp.float32)
        h = jnp.maximum(mm(ea_ref[...], W1_ref[...]) + b1_ref[...], 0.0)
        We = mm(h.astype(jnp.bfloat16), W2_ref[...]) + b2_ref[...]
        xr = mm(xj_ref[...], R_ref[...])
        msg = mm((xr * We).astype(jnp.bfloat16), S_ref[...])

        ohb = gb_ref[:, 0:1] == lax.broadcasted_iota(jnp.int32, (1, _G), 1)
        if valid_limit < _EPH:
            eid = i * _EB + lax.broadcasted_iota(jnp.int32, (_EB, 1), 0)
            ohb = ohb & (eid < valid_limit)
        oh = ohb.astype(jnp.bfloat16)
        out_ref[...] += lax.dot_general(
            oh, msg.astype(jnp.bfloat16), (((0,), (0,)), ((), ())),
            preferred_element_type=jnp.float32)
    return body


def _tc_edge(valid_limit, ea_h, xj_h, gb_h, W1, b1, W2, b2, R, S):
    full = lambda shape: pl.BlockSpec(shape, lambda i: (0,) * len(shape))
    return pl.pallas_call(
        _make_tc_edge(valid_limit),
        grid=(_GRIDH,),
        in_specs=[
            pl.BlockSpec((_EB, _F_EDGE), lambda i: (i, 0)),
            pl.BlockSpec((_EB, _F_IN), lambda i: (i, 0)),
            pl.BlockSpec((_EB, 8), lambda i: (i, 0)),
            full((_F_EDGE, _EDGE_H)),
            full((1, _EDGE_H)),
            full((_EDGE_H, _F_IN * _C_OUT)),
            full((1, _F_IN * _C_OUT)),
            full((_F_IN, _F_IN * _C_OUT)),
            full((_F_IN * _C_OUT, _C_OUT)),
        ],
        out_specs=full((_G, _C_OUT)),
        out_shape=jax.ShapeDtypeStruct((_G, _C_OUT), jnp.float32),
    )(ea_h, xj_h, gb_h, W1, b1, W2, b2, R, S)


def _tc_final_body(*refs):
    p_refs = refs[:_NH]
    (na_ref, bat_ref, Wr_ref, bc_ref,
     Wd1_ref, bd1_ref, Wd2_ref, bd2_ref, out_ref) = refs[_NH:]
    ohB = (bat_ref[...] == lax.broadcasted_iota(
        jnp.int32, (1, _G), 1)).astype(jnp.float32)
    t = lax.dot_general(ohB, na_ref[...],
                        (((0,), (0,)), ((), ())), precision=_DP)
    gnode = jnp.dot(t, Wr_ref[...], precision=_DP)
    cnt = lax.dot_general(ohB, jnp.ones((_N, 1), jnp.float32),
                          (((0,), (0,)), ((), ())), precision=_DP)
    g = gnode + cnt * bc_ref[...]
    for p_ref in p_refs:
        g = g + p_ref[...]
    gr = jnp.maximum(
        jnp.dot(g, Wd1_ref[...], precision=_DP) + bd1_ref[...], 0.0)
    out_ref[...] = jnp.dot(gr, Wd2_ref[...], precision=_DP) + bd2_ref[...]


def _tc_final(parts, node_attr, bat2, W_root, b_conv, Wd1, bd1, Wd2, bd2):
    return pl.pallas_call(
        _tc_final_body,
        out_shape=jax.ShapeDtypeStruct((_G, _D2), jnp.float32),
    )(*parts, node_attr, bat2, W_root, b_conv, Wd1, bd1, Wd2, bd2)


def kernel(node_attr, edge_index, edge_attr, batching, W1, b1, W2, b2,
           W_root, b_conv, Wd1, bd1, Wd2, bd2):
    pad = _EP - _E
    src2d = jnp.concatenate(
        [edge_index[0], jnp.zeros((pad,), jnp.int32)]).reshape(_NW * _CPW, _CH)
    dst2d = jnp.concatenate(
        [edge_index[1], jnp.zeros((pad,), jnp.int32)]).reshape(_NW * _CPW, _CH)
    sd2d = jnp.concatenate([src2d, dst2d], axis=1)
    ea_p = jnp.concatenate(
        [edge_attr, jnp.zeros((pad, _F_EDGE), jnp.float32)]).astype(
            jnp.bfloat16)
    bat8 = jnp.broadcast_to(batching[:, None], (_N, 8))
    nab = node_attr.astype(jnp.bfloat16)

    R_np = np.zeros((_F_IN, _F_IN * _C_OUT), np.float32)
    for i in range(_F_IN):
        R_np[i, i * _C_OUT:(i + 1) * _C_OUT] = 1.0
    S_np = np.kron(np.ones((_F_IN, 1), np.float32),
                   np.eye(_C_OUT, dtype=np.float32))
    R = jnp.asarray(R_np).astype(jnp.bfloat16)
    S = jnp.asarray(S_np).astype(jnp.bfloat16)
    W1b = W1.astype(jnp.bfloat16)
    W2b = W2.astype(jnp.bfloat16)

    b1r = b1.reshape(1, -1)
    b2r = b2.reshape(1, -1)
    rows_h = _NW * _CPWH
    parts = []
    for hh in range(_NH):
        sd_h = lax.slice_in_dim(sd2d, hh * rows_h, (hh + 1) * rows_h, axis=0)
        xj_h, gb_h = _sc_gather(_CPWH, nab, bat8, sd_h)
        ea_h = lax.slice_in_dim(ea_p, hh * _EPH, (hh + 1) * _EPH, axis=0)
        limit = min(_EPH, max(0, _E - hh * _EPH))
        parts.append(_tc_edge(limit, ea_h, xj_h, gb_h, W1b, b1r, W2b, b2r,
                              R, S))

    return _tc_final(parts, node_attr, batching.reshape(_N, 1),
                     W_root, b_conv.reshape(1, -1), Wd1, bd1.reshape(1, -1),
                     Wd2, bd2.reshape(1, -1))
